# baseline (speedup 1.0000x reference)
import jax
import jax.numpy as jnp
from jax.experimental import pallas as pl
from jax.experimental.pallas import tpu as pltpu

E = 8
TOPK = 2
D = 768
FF = 4 * 768
T = 2048
FB = 768
NF = FF // FB


def _moe_body(x_ref, wr_ref, wfc_ref, wproj_ref, wsfc_ref, wsproj_ref,
              out_ref, w_ref):
    e = pl.program_id(0)
    f = pl.program_id(1)
    xx = x_ref[...]

    @pl.when(jnp.logical_and(e == 0, f == 0))
    def _router():
        logits = jax.lax.dot_general(
            xx, wr_ref[...], (((1,), (1,)), ((), ())),
            preferred_element_type=jnp.float32)
        m = jnp.max(logits, axis=-1, keepdims=True)
        p = jnp.exp(logits - m)
        p = p / jnp.sum(p, axis=-1, keepdims=True)
        lane = jax.lax.broadcasted_iota(jnp.int32, (T, E), 1)
        p1 = jnp.max(p, axis=-1, keepdims=True)
        a1 = jnp.min(jnp.where(p == p1, lane, E), axis=-1, keepdims=True)
        mask1 = lane == a1
        pm = jnp.where(mask1, -jnp.inf, p)
        p2 = jnp.max(pm, axis=-1, keepdims=True)
        a2 = jnp.min(jnp.where(pm == p2, lane, E), axis=-1, keepdims=True)
        mask2 = lane == a2
        denom = jnp.maximum(p1 + p2, 1e-9)
        w = jnp.where(mask1, p1, 0.0) + jnp.where(mask2, p2, 0.0)
        w_ref[...] = w / denom

    @pl.when(e == 0)
    def _shared():
        hs = jax.lax.dot_general(
            xx, wsfc_ref[...], (((1,), (1,)), ((), ())),
            preferred_element_type=jnp.float32)
        a_s = jnp.square(jnp.maximum(hs, 0.0))
        ys = jax.lax.dot_general(
            a_s, wsproj_ref[...], (((1,), (1,)), ((), ())),
            preferred_element_type=jnp.float32)

        @pl.when(f == 0)
        def _():
            out_ref[...] = ys

        @pl.when(f != 0)
        def _():
            out_ref[...] += ys

    h = jax.lax.dot_general(
        xx, wfc_ref[0], (((1,), (1,)), ((), ())),
        preferred_element_type=jnp.float32)
    a = jnp.square(jnp.maximum(h, 0.0))
    y = jax.lax.dot_general(
        a, wproj_ref[0], (((1,), (1,)), ((), ())),
        preferred_element_type=jnp.float32)
    lane = jax.lax.broadcasted_iota(jnp.int32, (T, E), 1)
    wcol = jnp.sum(jnp.where(lane == e, w_ref[...], 0.0), axis=-1,
                   keepdims=True)
    out_ref[...] += wcol * y


def kernel(x, W_router, Wfc, Wproj, Wsfc, Wsproj):
    B, S, Dd = x.shape
    flat = x.reshape(B * S, Dd)
    out = pl.pallas_call(
        _moe_body,
        grid=(E, NF),
        in_specs=[
            pl.BlockSpec((T, D), lambda e, f: (0, 0)),
            pl.BlockSpec((E, D), lambda e, f: (0, 0)),
            pl.BlockSpec((1, FB, D), lambda e, f: (e, f, 0)),
            pl.BlockSpec((1, D, FB), lambda e, f: (e, 0, f)),
            pl.BlockSpec((FB, D), lambda e, f: (f, 0)),
            pl.BlockSpec((D, FB), lambda e, f: (0, f)),
        ],
        out_specs=pl.BlockSpec((T, D), lambda e, f: (0, 0)),
        out_shape=jax.ShapeDtypeStruct((T, D), jnp.float32),
        scratch_shapes=[pltpu.VMEM((T, E), jnp.float32)],
    )(flat, W_router, Wfc, Wproj, Wsfc, Wsproj)
    return out.reshape(B, S, Dd)
